# baseline (speedup 1.0000x reference)
import functools

import jax
import jax.numpy as jnp
from jax import lax
from jax.experimental import pallas as pl
from jax.experimental.pallas import tpu as pltpu
from jax.experimental.pallas import tpu_sc as plsc

NUM_USERS = 1_000_000
NUM_ITEMS = 100_000
DIM_FEAT = 512
FACTORS = 64
B = 16384

NC = 2
NS = 16
NW = NC * NS
B_PER_W = B // NW
FCHUNK = 64
NFCHUNK = B_PER_W // FCHUNK


def _sc_gather_body(user_hbm, item_i_hbm, item_j_hbm,
                    user_tab, item_tab, feat_tab,
                    u_out, ii_out, ij_out, fi_out, fj_out,
                    idx_u, idx_i, idx_j, emb_buf, fbuf0, fbuf1,
                    sem_e, sem_f0, sem_f1):
  wid = lax.axis_index("s") * NC + lax.axis_index("c")
  base = wid * B_PER_W

  pltpu.sync_copy(user_hbm.at[pl.ds(base, B_PER_W)], idx_u)
  pltpu.sync_copy(item_i_hbm.at[pl.ds(base, B_PER_W)], idx_i)
  pltpu.sync_copy(item_j_hbm.at[pl.ds(base, B_PER_W)], idx_j)

  pltpu.async_copy(user_tab.at[idx_u], emb_buf, sem_e).wait()
  pltpu.sync_copy(emb_buf, u_out.at[pl.ds(base, B_PER_W)])
  pltpu.async_copy(item_tab.at[idx_i], emb_buf, sem_e).wait()
  pltpu.sync_copy(emb_buf, ii_out.at[pl.ds(base, B_PER_W)])
  pltpu.async_copy(item_tab.at[idx_j], emb_buf, sem_e).wait()
  pltpu.sync_copy(emb_buf, ij_out.at[pl.ds(base, B_PER_W)])

  for idx, out in ((idx_i, fi_out), (idx_j, fj_out)):
    for k in range(NFCHUNK):
      fbuf = fbuf0 if (k % 2 == 0) else fbuf1
      sem = sem_f0 if (k % 2 == 0) else sem_f1
      pltpu.async_copy(feat_tab.at[idx.at[pl.ds(k * FCHUNK, FCHUNK)]],
                       fbuf, sem).wait()
      pltpu.sync_copy(fbuf, out.at[pl.ds(base + k * FCHUNK, FCHUNK)])


@jax.jit
def _sc_gather(user, item_i, item_j, user_tab, item_tab, feat_tab):
  mesh = plsc.VectorSubcoreMesh(core_axis_name="c", subcore_axis_name="s")
  f32 = jnp.float32
  return pl.kernel(
      _sc_gather_body,
      out_type=(
          jax.ShapeDtypeStruct((B, FACTORS), f32),
          jax.ShapeDtypeStruct((B, FACTORS), f32),
          jax.ShapeDtypeStruct((B, FACTORS), f32),
          jax.ShapeDtypeStruct((B, DIM_FEAT), f32),
          jax.ShapeDtypeStruct((B, DIM_FEAT), f32),
      ),
      mesh=mesh,
      scratch_types=(
          pltpu.VMEM((B_PER_W,), jnp.int32),
          pltpu.VMEM((B_PER_W,), jnp.int32),
          pltpu.VMEM((B_PER_W,), jnp.int32),
          pltpu.VMEM((B_PER_W, FACTORS), f32),
          pltpu.VMEM((FCHUNK, DIM_FEAT), f32),
          pltpu.VMEM((FCHUNK, DIM_FEAT), f32),
          pltpu.SemaphoreType.DMA,
          pltpu.SemaphoreType.DMA,
          pltpu.SemaphoreType.DMA,
      ),
      compiler_params=pltpu.CompilerParams(use_tc_tiling_on_sc=False),
      name="vbpr_sc_gather",
  )(user, item_i, item_j, user_tab, item_tab, feat_tab)


BM = 2048


def _tc_dense_body(u_ref, ii_ref, ij_ref, fi_ref, fj_ref,
                   wv_ref, watt_ref, bvis_ref, wvb_ref, scal_ref, out_ref):
  u = u_ref[...]
  wv = wv_ref[...]
  watt = watt_ref[...]
  bvis = bvis_ref[...]
  wvb = wvb_ref[...]
  b_vbias = scal_ref[0, 0]
  b_att = scal_ref[0, 1]

  def score(item_emb, feat):
    vis = lax.dot_general(feat, wv, (((1,), (1,)), ((), ())),
                          preferred_element_type=jnp.float32) + bvis
    a_item = jnp.tanh(jnp.sum(item_emb * watt, axis=1) + b_att)
    a_vis = jnp.tanh(jnp.sum(vis * watt, axis=1) + b_att)
    e_item = jnp.exp(a_item)
    e_vis = jnp.exp(a_vis)
    denom = e_item + e_vis
    d_item = jnp.sum(u * item_emb, axis=1)
    d_vis = jnp.sum(u * vis, axis=1)
    featb = jnp.sum(feat * wvb, axis=1)
    return (e_item * d_item + e_vis * d_vis) / denom + featb + b_vbias

  out_ref[...] = (score(ii_ref[...], fi_ref[...])
                  - score(ij_ref[...], fj_ref[...]))[:, None]


@jax.jit
def _tc_dense(u_emb, ii_emb, ij_emb, fi, fj, W_vis, w_att, b_vis, w_vbias,
              scalars):
  grid = (B // BM,)
  row = lambda i: (i, 0)
  fixed = lambda i: (0, 0)
  out = pl.pallas_call(
      _tc_dense_body,
      grid=grid,
      in_specs=[
          pl.BlockSpec((BM, FACTORS), row),
          pl.BlockSpec((BM, FACTORS), row),
          pl.BlockSpec((BM, FACTORS), row),
          pl.BlockSpec((BM, DIM_FEAT), row),
          pl.BlockSpec((BM, DIM_FEAT), row),
          pl.BlockSpec((FACTORS, DIM_FEAT), fixed),
          pl.BlockSpec((1, FACTORS), fixed),
          pl.BlockSpec((1, FACTORS), fixed),
          pl.BlockSpec((1, DIM_FEAT), fixed),
          pl.BlockSpec((1, 2), fixed),
      ],
      out_specs=pl.BlockSpec((BM, 1), row),
      out_shape=jax.ShapeDtypeStruct((B, 1), jnp.float32),
      name="vbpr_tc_dense",
  )(u_emb, ii_emb, ij_emb, fi, fj, W_vis, w_att, b_vis, w_vbias, scalars)
  return out.reshape(B)


def kernel(user, item_i, item_j, user_table, item_table, item_features,
           W_vis, b_vis, w_vbias, b_vbias, w_att, b_att):
  user = user.astype(jnp.int32)
  item_i = item_i.astype(jnp.int32)
  item_j = item_j.astype(jnp.int32)
  u_emb, ii_emb, ij_emb, fi, fj = _sc_gather(
      user, item_i, item_j, user_table, item_table, item_features)
  scalars = jnp.stack([b_vbias, b_att]).reshape(1, 2).astype(jnp.float32)
  return _tc_dense(u_emb, ii_emb, ij_emb, fi, fj,
                   W_vis, w_att.reshape(1, FACTORS),
                   b_vis.reshape(1, FACTORS), w_vbias.reshape(1, DIM_FEAT),
                   scalars)


# speedup vs baseline: 1.1821x; 1.1821x over previous
import jax
import jax.numpy as jnp
from jax import lax
from jax.experimental import pallas as pl
from jax.experimental.pallas import tpu as pltpu
from jax.experimental.pallas import tpu_sc as plsc

NUM_USERS = 1_000_000
NUM_ITEMS = 100_000
DIM_FEAT = 512
FACTORS = 64
B = 16384

NC = 2
NS = 16
NW = NC * NS
B_PER_W = B // NW
FCHUNK = 64
NFCHUNK = B_PER_W // FCHUNK
ECHUNK = 128
NECHUNK = B_PER_W // ECHUNK


def _sc_gather_body(user_hbm, item_i_hbm, item_j_hbm,
                    item_i_full_hbm, item_j_full_hbm,
                    user_tab2, item_tab2, feat_tab,
                    u_out, ii_out, ij_out, fi_out, fj_out,
                    idx_u, idx_i, idx_j, idx_if, idx_jf, eb0, eb1, fb0, fb1,
                    sem_g0, sem_g1, sem_s0, sem_s1):
  wid = lax.axis_index("s") * NC + lax.axis_index("c")
  base = wid * B_PER_W

  pltpu.sync_copy(user_hbm.at[pl.ds(base, B_PER_W)], idx_u)
  pltpu.sync_copy(item_i_hbm.at[pl.ds(base, B_PER_W)], idx_i)
  pltpu.sync_copy(item_j_hbm.at[pl.ds(base, B_PER_W)], idx_j)
  pltpu.sync_copy(item_i_full_hbm.at[pl.ds(base, B_PER_W)], idx_if)
  pltpu.sync_copy(item_j_full_hbm.at[pl.ds(base, B_PER_W)], idx_jf)

  ebufs = (eb0, eb1)
  fbufs = (fb0, fb1)
  gsems = (sem_g0, sem_g1)
  ssems = (sem_s0, sem_s1)

  echunks = []
  for idx, out in ((idx_u, u_out), (idx_i, ii_out), (idx_j, ij_out)):
    for k in range(NECHUNK):
      echunks.append((idx, k * ECHUNK, ECHUNK, user_tab2 if out is u_out
                      else item_tab2, out))
  fchunks = []
  for idx, out in ((idx_if, fi_out), (idx_jf, fj_out)):
    for k in range(NFCHUNK):
      fchunks.append((idx, k * FCHUNK, FCHUNK, feat_tab, out))

  def run(chunks, bufs):
    n = len(chunks)
    gathers = [None] * n
    stores = [None] * n
    for k, (idx, off, rows, tab, out) in enumerate(chunks):
      b = k % 2
      if k >= 2:
        stores[k - 2].wait()
      gathers[k] = pltpu.async_copy(tab.at[idx.at[pl.ds(off, rows)]],
                                    bufs[b], gsems[b])
      gathers[k].wait()
      stores[k] = pltpu.async_copy(bufs[b], out.at[pl.ds(base + off, rows)],
                                   ssems[b])
    for k in range(max(0, n - 2), n):
      stores[k].wait()

  run(echunks, ebufs)
  run(fchunks, fbufs)


@jax.jit
def _sc_gather(user, item_i, item_j, item_i_full, item_j_full,
               user_tab2, item_tab2, feat_tab):
  mesh = plsc.VectorSubcoreMesh(core_axis_name="c", subcore_axis_name="s")
  f32 = jnp.float32
  return pl.kernel(
      _sc_gather_body,
      out_type=(
          jax.ShapeDtypeStruct((B, 2 * FACTORS), f32),
          jax.ShapeDtypeStruct((B, 2 * FACTORS), f32),
          jax.ShapeDtypeStruct((B, 2 * FACTORS), f32),
          jax.ShapeDtypeStruct((B, DIM_FEAT), f32),
          jax.ShapeDtypeStruct((B, DIM_FEAT), f32),
      ),
      mesh=mesh,
      scratch_types=(
          pltpu.VMEM((B_PER_W,), jnp.int32),
          pltpu.VMEM((B_PER_W,), jnp.int32),
          pltpu.VMEM((B_PER_W,), jnp.int32),
          pltpu.VMEM((B_PER_W,), jnp.int32),
          pltpu.VMEM((B_PER_W,), jnp.int32),
          pltpu.VMEM((ECHUNK, 2 * FACTORS), f32),
          pltpu.VMEM((ECHUNK, 2 * FACTORS), f32),
          pltpu.VMEM((FCHUNK, DIM_FEAT), f32),
          pltpu.VMEM((FCHUNK, DIM_FEAT), f32),
          pltpu.SemaphoreType.DMA,
          pltpu.SemaphoreType.DMA,
          pltpu.SemaphoreType.DMA,
          pltpu.SemaphoreType.DMA,
      ),
      name="vbpr_sc_gather",
  )(user, item_i, item_j, item_i_full, item_j_full,
    user_tab2, item_tab2, feat_tab)


BM = 2048


def _tc_dense_body(u2_ref, ii2_ref, ij2_ref, fi_ref, fj_ref, par_ref,
                   wv_ref, watt_ref, bvis_ref, wvb_ref, scal_ref, out_ref):
  wv = wv_ref[...]
  watt = watt_ref[...]
  bvis = bvis_ref[...]
  wvb = wvb_ref[...]
  b_vbias = scal_ref[0, 0]
  b_att = scal_ref[0, 1]

  def half(ref, col):
    two = ref[...]
    par = par_ref[:, col][:, None]
    return jnp.where(par > 0, two[:, FACTORS:], two[:, :FACTORS])

  u = half(u2_ref, 0)

  def score(item_emb, feat):
    vis = lax.dot_general(feat, wv, (((1,), (1,)), ((), ())),
                          preferred_element_type=jnp.float32) + bvis
    a_item = jnp.tanh(jnp.sum(item_emb * watt, axis=1) + b_att)
    a_vis = jnp.tanh(jnp.sum(vis * watt, axis=1) + b_att)
    e_item = jnp.exp(a_item)
    e_vis = jnp.exp(a_vis)
    denom = e_item + e_vis
    d_item = jnp.sum(u * item_emb, axis=1)
    d_vis = jnp.sum(u * vis, axis=1)
    featb = jnp.sum(feat * wvb, axis=1)
    return (e_item * d_item + e_vis * d_vis) / denom + featb + b_vbias

  out_ref[...] = (score(half(ii2_ref, 1), fi_ref[...])
                  - score(half(ij2_ref, 2), fj_ref[...]))[:, None]


@jax.jit
def _tc_dense(u2, ii2, ij2, fi, fj, par, W_vis, w_att, b_vis, w_vbias,
              scalars):
  grid = (B // BM,)
  row = lambda i: (i, 0)
  fixed = lambda i: (0, 0)
  out = pl.pallas_call(
      _tc_dense_body,
      grid=grid,
      in_specs=[
          pl.BlockSpec((BM, 2 * FACTORS), row),
          pl.BlockSpec((BM, 2 * FACTORS), row),
          pl.BlockSpec((BM, 2 * FACTORS), row),
          pl.BlockSpec((BM, DIM_FEAT), row),
          pl.BlockSpec((BM, DIM_FEAT), row),
          pl.BlockSpec((BM, 3), row),
          pl.BlockSpec((FACTORS, DIM_FEAT), fixed),
          pl.BlockSpec((1, FACTORS), fixed),
          pl.BlockSpec((1, FACTORS), fixed),
          pl.BlockSpec((1, DIM_FEAT), fixed),
          pl.BlockSpec((1, 2), fixed),
      ],
      out_specs=pl.BlockSpec((BM, 1), row),
      out_shape=jax.ShapeDtypeStruct((B, 1), jnp.float32),
      name="vbpr_tc_dense",
  )(u2, ii2, ij2, fi, fj, par, W_vis, w_att, b_vis, w_vbias, scalars)
  return out.reshape(B)


def kernel(user, item_i, item_j, user_table, item_table, item_features,
           W_vis, b_vis, w_vbias, b_vbias, w_att, b_att):
  user = user.astype(jnp.int32)
  item_i = item_i.astype(jnp.int32)
  item_j = item_j.astype(jnp.int32)
  user_tab2 = user_table.reshape(NUM_USERS // 2, 2 * FACTORS)
  item_tab2 = item_table.reshape(NUM_ITEMS // 2, 2 * FACTORS)
  par = jnp.stack([user & 1, item_i & 1, item_j & 1],
                  axis=1).astype(jnp.float32)
  u2, ii2, ij2, fi, fj = _sc_gather(
      user >> 1, item_i >> 1, item_j >> 1, item_i, item_j,
      user_tab2, item_tab2, item_features)
  scalars = jnp.stack([b_vbias, b_att]).reshape(1, 2).astype(jnp.float32)
  return _tc_dense(u2, ii2, ij2, fi, fj, par,
                   W_vis, w_att.reshape(1, FACTORS),
                   b_vis.reshape(1, FACTORS), w_vbias.reshape(1, DIM_FEAT),
                   scalars)


# speedup vs baseline: 1.2331x; 1.0432x over previous
import jax
import jax.numpy as jnp
from jax import lax
from jax.experimental import pallas as pl
from jax.experimental.pallas import tpu as pltpu
from jax.experimental.pallas import tpu_sc as plsc

NUM_USERS = 1_000_000
NUM_ITEMS = 100_000
DIM_FEAT = 512
FACTORS = 64
B = 16384

NC = 2
NS = 16
NW = NC * NS
B_PER_W = B // NW
FCHUNK = 64
NFCHUNK = B_PER_W // FCHUNK


def _sc_feat_body(item_i_hbm, item_j_hbm, feat_tab,
                  fi_out, fj_out,
                  idx_i, idx_j, fb0, fb1,
                  sem_g0, sem_g1, sem_s0, sem_s1):
  wid = lax.axis_index("s") * NC + lax.axis_index("c")
  base = wid * B_PER_W

  pltpu.sync_copy(item_i_hbm.at[pl.ds(base, B_PER_W)], idx_i)
  pltpu.sync_copy(item_j_hbm.at[pl.ds(base, B_PER_W)], idx_j)

  bufs = (fb0, fb1)
  gsems = (sem_g0, sem_g1)
  ssems = (sem_s0, sem_s1)

  chunks = []
  for idx, out in ((idx_i, fi_out), (idx_j, fj_out)):
    for k in range(NFCHUNK):
      chunks.append((idx, k * FCHUNK, out))

  n = len(chunks)
  stores = [None] * n
  for k, (idx, off, out) in enumerate(chunks):
    b = k % 2
    if k >= 2:
      stores[k - 2].wait()
    g = pltpu.async_copy(feat_tab.at[idx.at[pl.ds(off, FCHUNK)]],
                         bufs[b], gsems[b])
    g.wait()
    stores[k] = pltpu.async_copy(bufs[b], out.at[pl.ds(base + off, FCHUNK)],
                                 ssems[b])
  for k in range(max(0, n - 2), n):
    stores[k].wait()


def _sc_narrow_body(user_hbm, item_i_hbm, item_j_hbm, user_tab, item_tab,
                    u_out, ii_out, ij_out,
                    idx_u, idx_i, idx_j, eb0, eb1, eb2,
                    sem_e0, sem_e1, sem_e2, sem_s0, sem_s1, sem_s2):
  wid = lax.axis_index("s") * NC + lax.axis_index("c")
  base = wid * B_PER_W

  pltpu.sync_copy(user_hbm.at[pl.ds(base, B_PER_W)], idx_u)
  pltpu.sync_copy(item_i_hbm.at[pl.ds(base, B_PER_W)], idx_i)
  pltpu.sync_copy(item_j_hbm.at[pl.ds(base, B_PER_W)], idx_j)

  gu = pltpu.async_copy(user_tab.at[idx_u], eb0, sem_e0)
  gi = pltpu.async_copy(item_tab.at[idx_i], eb1, sem_e1)
  gj = pltpu.async_copy(item_tab.at[idx_j], eb2, sem_e2)
  gu.wait()
  su = pltpu.async_copy(eb0, u_out.at[pl.ds(base, B_PER_W)], sem_s0)
  gi.wait()
  si = pltpu.async_copy(eb1, ii_out.at[pl.ds(base, B_PER_W)], sem_s1)
  gj.wait()
  sj = pltpu.async_copy(eb2, ij_out.at[pl.ds(base, B_PER_W)], sem_s2)
  su.wait()
  si.wait()
  sj.wait()


_MESH = plsc.VectorSubcoreMesh(core_axis_name="c", subcore_axis_name="s")


@jax.jit
def _sc_gather(user, item_i, item_j, user_tab, item_tab, feat_tab):
  f32 = jnp.float32
  fi, fj = pl.kernel(
      _sc_feat_body,
      out_type=(
          jax.ShapeDtypeStruct((B, DIM_FEAT), f32),
          jax.ShapeDtypeStruct((B, DIM_FEAT), f32),
      ),
      mesh=_MESH,
      scratch_types=(
          pltpu.VMEM((B_PER_W,), jnp.int32),
          pltpu.VMEM((B_PER_W,), jnp.int32),
          pltpu.VMEM((FCHUNK, DIM_FEAT), f32),
          pltpu.VMEM((FCHUNK, DIM_FEAT), f32),
          pltpu.SemaphoreType.DMA,
          pltpu.SemaphoreType.DMA,
          pltpu.SemaphoreType.DMA,
          pltpu.SemaphoreType.DMA,
      ),
      name="vbpr_sc_feat",
  )(item_i, item_j, feat_tab)

  u_emb, ii_emb, ij_emb = pl.kernel(
      _sc_narrow_body,
      out_type=(
          jax.ShapeDtypeStruct((B, FACTORS), f32),
          jax.ShapeDtypeStruct((B, FACTORS), f32),
          jax.ShapeDtypeStruct((B, FACTORS), f32),
      ),
      mesh=_MESH,
      scratch_types=(
          pltpu.VMEM((B_PER_W,), jnp.int32),
          pltpu.VMEM((B_PER_W,), jnp.int32),
          pltpu.VMEM((B_PER_W,), jnp.int32),
          pltpu.VMEM((B_PER_W, FACTORS), f32),
          pltpu.VMEM((B_PER_W, FACTORS), f32),
          pltpu.VMEM((B_PER_W, FACTORS), f32),
          pltpu.SemaphoreType.DMA,
          pltpu.SemaphoreType.DMA,
          pltpu.SemaphoreType.DMA,
          pltpu.SemaphoreType.DMA,
          pltpu.SemaphoreType.DMA,
          pltpu.SemaphoreType.DMA,
      ),
      compiler_params=pltpu.CompilerParams(use_tc_tiling_on_sc=False),
      name="vbpr_sc_narrow",
  )(user, item_i, item_j, user_tab, item_tab)

  return u_emb, ii_emb, ij_emb, fi, fj


BM = 2048


def _tc_dense_body(u_ref, ii_ref, ij_ref, fi_ref, fj_ref,
                   wv_ref, watt_ref, bvis_ref, wvb_ref, scal_ref, out_ref):
  u = u_ref[...]
  wv = wv_ref[...]
  watt = watt_ref[...]
  bvis = bvis_ref[...]
  wvb = wvb_ref[...]
  b_vbias = scal_ref[0, 0]
  b_att = scal_ref[0, 1]

  def score(item_emb, feat):
    vis = lax.dot_general(feat, wv, (((1,), (1,)), ((), ())),
                          preferred_element_type=jnp.float32) + bvis
    a_item = jnp.tanh(jnp.sum(item_emb * watt, axis=1) + b_att)
    a_vis = jnp.tanh(jnp.sum(vis * watt, axis=1) + b_att)
    e_item = jnp.exp(a_item)
    e_vis = jnp.exp(a_vis)
    denom = e_item + e_vis
    d_item = jnp.sum(u * item_emb, axis=1)
    d_vis = jnp.sum(u * vis, axis=1)
    featb = jnp.sum(feat * wvb, axis=1)
    return (e_item * d_item + e_vis * d_vis) / denom + featb + b_vbias

  out_ref[...] = (score(ii_ref[...], fi_ref[...])
                  - score(ij_ref[...], fj_ref[...]))[:, None]


@jax.jit
def _tc_dense(u_emb, ii_emb, ij_emb, fi, fj, W_vis, w_att, b_vis, w_vbias,
              scalars):
  grid = (B // BM,)
  row = lambda i: (i, 0)
  fixed = lambda i: (0, 0)
  out = pl.pallas_call(
      _tc_dense_body,
      grid=grid,
      in_specs=[
          pl.BlockSpec((BM, FACTORS), row),
          pl.BlockSpec((BM, FACTORS), row),
          pl.BlockSpec((BM, FACTORS), row),
          pl.BlockSpec((BM, DIM_FEAT), row),
          pl.BlockSpec((BM, DIM_FEAT), row),
          pl.BlockSpec((FACTORS, DIM_FEAT), fixed),
          pl.BlockSpec((1, FACTORS), fixed),
          pl.BlockSpec((1, FACTORS), fixed),
          pl.BlockSpec((1, DIM_FEAT), fixed),
          pl.BlockSpec((1, 2), fixed),
      ],
      out_specs=pl.BlockSpec((BM, 1), row),
      out_shape=jax.ShapeDtypeStruct((B, 1), jnp.float32),
      name="vbpr_tc_dense",
  )(u_emb, ii_emb, ij_emb, fi, fj, W_vis, w_att, b_vis, w_vbias, scalars)
  return out.reshape(B)


def kernel(user, item_i, item_j, user_table, item_table, item_features,
           W_vis, b_vis, w_vbias, b_vbias, w_att, b_att):
  user = user.astype(jnp.int32)
  item_i = item_i.astype(jnp.int32)
  item_j = item_j.astype(jnp.int32)
  u_emb, ii_emb, ij_emb, fi, fj = _sc_gather(
      user, item_i, item_j, user_table, item_table, item_features)
  scalars = jnp.stack([b_vbias, b_att]).reshape(1, 2).astype(jnp.float32)
  return _tc_dense(u_emb, ii_emb, ij_emb, fi, fj,
                   W_vis, w_att.reshape(1, FACTORS),
                   b_vis.reshape(1, FACTORS), w_vbias.reshape(1, DIM_FEAT),
                   scalars)


# speedup vs baseline: 1.8479x; 1.4986x over previous
import jax
import jax.numpy as jnp
from jax import lax
from jax.experimental import pallas as pl
from jax.experimental.pallas import tpu as pltpu
from jax.experimental.pallas import tpu_sc as plsc

NUM_USERS = 1_000_000
NUM_ITEMS = 100_000
DIM_FEAT = 512
FACTORS = 64
B = 16384

NC = 2
NS = 16
NW = NC * NS
B_PER_W = B // NW
FCHUNK = 64
NFCHUNK = B_PER_W // FCHUNK


def _sc_feat_body(item_i_hbm, item_j_hbm, feat_tab,
                  fi_out, fj_out,
                  idx_i, idx_j, fb0, fb1,
                  sem_g0, sem_g1, sem_s0, sem_s1):
  wid = lax.axis_index("s") * NC + lax.axis_index("c")
  base = wid * B_PER_W

  pltpu.sync_copy(item_i_hbm.at[pl.ds(base, B_PER_W)], idx_i)
  pltpu.sync_copy(item_j_hbm.at[pl.ds(base, B_PER_W)], idx_j)

  bufs = (fb0, fb1)
  gsems = (sem_g0, sem_g1)
  ssems = (sem_s0, sem_s1)

  chunks = []
  for idx, out in ((idx_i, fi_out), (idx_j, fj_out)):
    for k in range(NFCHUNK):
      chunks.append((idx, k * FCHUNK, out))

  n = len(chunks)
  stores = [None] * n
  for k, (idx, off, out) in enumerate(chunks):
    b = k % 2
    if k >= 2:
      stores[k - 2].wait()
    g = pltpu.async_copy(feat_tab.at[idx.at[pl.ds(off, FCHUNK)]],
                         bufs[b], gsems[b])
    g.wait()
    stores[k] = pltpu.async_copy(bufs[b], out.at[pl.ds(base + off, FCHUNK)],
                                 ssems[b])
  for k in range(max(0, n - 2), n):
    stores[k].wait()


def _sc_narrow_body(idx_hbm, tab, out,
                    idx_v, eb0, sem):
  wid = lax.axis_index("s") * NC + lax.axis_index("c")
  base = wid * B_PER_W

  pltpu.sync_copy(idx_hbm.at[pl.ds(base, B_PER_W)], idx_v)

  def body(g, carry):
    v = idx_v[pl.ds(g * 16, 16)]
    for l in range(16):
      pltpu.async_copy(tab.at[pl.ds(v[l], 1)],
                       eb0.at[pl.ds(g * 16 + l, 1)], sem)
    return carry

  lax.fori_loop(0, B_PER_W // 16, body, 0)
  pltpu.make_async_copy(tab.at[pl.ds(0, B_PER_W)], eb0, sem).wait()
  pltpu.sync_copy(eb0, out.at[pl.ds(base, B_PER_W)])


_MESH = plsc.VectorSubcoreMesh(core_axis_name="c", subcore_axis_name="s")


@jax.jit
def _sc_gather(user, item_i, item_j, user_tab, item_tab, feat_tab):
  f32 = jnp.float32
  fi, fj = pl.kernel(
      _sc_feat_body,
      out_type=(
          jax.ShapeDtypeStruct((B, DIM_FEAT), f32),
          jax.ShapeDtypeStruct((B, DIM_FEAT), f32),
      ),
      mesh=_MESH,
      scratch_types=(
          pltpu.VMEM((B_PER_W,), jnp.int32),
          pltpu.VMEM((B_PER_W,), jnp.int32),
          pltpu.VMEM((FCHUNK, DIM_FEAT), f32),
          pltpu.VMEM((FCHUNK, DIM_FEAT), f32),
          pltpu.SemaphoreType.DMA,
          pltpu.SemaphoreType.DMA,
          pltpu.SemaphoreType.DMA,
          pltpu.SemaphoreType.DMA,
      ),
      name="vbpr_sc_feat",
  )(item_i, item_j, feat_tab)

  def narrow(idx, tab, tag):
    return pl.kernel(
        _sc_narrow_body,
        out_type=jax.ShapeDtypeStruct((B, FACTORS), f32),
        mesh=_MESH,
        scratch_types=(
            pltpu.VMEM((B_PER_W,), jnp.int32),
            pltpu.VMEM((B_PER_W, FACTORS), f32),
            pltpu.SemaphoreType.DMA,
        ),
        name="vbpr_sc_narrow_" + tag,
    )(idx, tab)

  u_emb = narrow(user, user_tab, "u")
  ii_emb = narrow(item_i, item_tab, "i")
  ij_emb = narrow(item_j, item_tab, "j")

  return u_emb, ii_emb, ij_emb, fi, fj


BM = 2048


def _tc_dense_body(u_ref, ii_ref, ij_ref, fi_ref, fj_ref,
                   wv_ref, watt_ref, bvis_ref, wvb_ref, scal_ref, out_ref):
  u = u_ref[...]
  wv = wv_ref[...]
  watt = watt_ref[...]
  bvis = bvis_ref[...]
  wvb = wvb_ref[...]
  b_vbias = scal_ref[0, 0]
  b_att = scal_ref[0, 1]

  def score(item_emb, feat):
    vis = lax.dot_general(feat, wv, (((1,), (1,)), ((), ())),
                          preferred_element_type=jnp.float32) + bvis
    a_item = jnp.tanh(jnp.sum(item_emb * watt, axis=1) + b_att)
    a_vis = jnp.tanh(jnp.sum(vis * watt, axis=1) + b_att)
    e_item = jnp.exp(a_item)
    e_vis = jnp.exp(a_vis)
    denom = e_item + e_vis
    d_item = jnp.sum(u * item_emb, axis=1)
    d_vis = jnp.sum(u * vis, axis=1)
    featb = jnp.sum(feat * wvb, axis=1)
    return (e_item * d_item + e_vis * d_vis) / denom + featb + b_vbias

  out_ref[...] = (score(ii_ref[...], fi_ref[...])
                  - score(ij_ref[...], fj_ref[...]))[:, None]


@jax.jit
def _tc_dense(u_emb, ii_emb, ij_emb, fi, fj, W_vis, w_att, b_vis, w_vbias,
              scalars):
  grid = (B // BM,)
  row = lambda i: (i, 0)
  fixed = lambda i: (0, 0)
  out = pl.pallas_call(
      _tc_dense_body,
      grid=grid,
      in_specs=[
          pl.BlockSpec((BM, FACTORS), row),
          pl.BlockSpec((BM, FACTORS), row),
          pl.BlockSpec((BM, FACTORS), row),
          pl.BlockSpec((BM, DIM_FEAT), row),
          pl.BlockSpec((BM, DIM_FEAT), row),
          pl.BlockSpec((FACTORS, DIM_FEAT), fixed),
          pl.BlockSpec((1, FACTORS), fixed),
          pl.BlockSpec((1, FACTORS), fixed),
          pl.BlockSpec((1, DIM_FEAT), fixed),
          pl.BlockSpec((1, 2), fixed),
      ],
      out_specs=pl.BlockSpec((BM, 1), row),
      out_shape=jax.ShapeDtypeStruct((B, 1), jnp.float32),
      name="vbpr_tc_dense",
  )(u_emb, ii_emb, ij_emb, fi, fj, W_vis, w_att, b_vis, w_vbias, scalars)
  return out.reshape(B)


def kernel(user, item_i, item_j, user_table, item_table, item_features,
           W_vis, b_vis, w_vbias, b_vbias, w_att, b_att):
  user = user.astype(jnp.int32)
  item_i = item_i.astype(jnp.int32)
  item_j = item_j.astype(jnp.int32)
  u_emb, ii_emb, ij_emb, fi, fj = _sc_gather(
      user, item_i, item_j, user_table, item_table, item_features)
  scalars = jnp.stack([b_vbias, b_att]).reshape(1, 2).astype(jnp.float32)
  return _tc_dense(u_emb, ii_emb, ij_emb, fi, fj,
                   W_vis, w_att.reshape(1, FACTORS),
                   b_vis.reshape(1, FACTORS), w_vbias.reshape(1, DIM_FEAT),
                   scalars)


# speedup vs baseline: 1.9456x; 1.0528x over previous
import jax
import jax.numpy as jnp
from jax import lax
from jax.experimental import pallas as pl
from jax.experimental.pallas import tpu as pltpu
from jax.experimental.pallas import tpu_sc as plsc

NUM_USERS = 1_000_000
NUM_ITEMS = 100_000
DIM_FEAT = 512
FACTORS = 64
B = 16384

NC = 2
NS = 16
NW = NC * NS
B_PER_W = B // NW
FCHUNK = 64
NFCHUNK = B_PER_W // FCHUNK


def _sc_feat_body(item_i_hbm, item_j_hbm, feat_tab,
                  fi_out, fj_out,
                  idx_i, idx_j, fb0, fb1,
                  sem_g0, sem_g1, sem_s0, sem_s1):
  wid = lax.axis_index("s") * NC + lax.axis_index("c")
  base = wid * B_PER_W

  pltpu.sync_copy(item_i_hbm.at[pl.ds(base, B_PER_W)], idx_i)
  pltpu.sync_copy(item_j_hbm.at[pl.ds(base, B_PER_W)], idx_j)

  bufs = (fb0, fb1)
  gsems = (sem_g0, sem_g1)
  ssems = (sem_s0, sem_s1)

  chunks = []
  for idx, out in ((idx_i, fi_out), (idx_j, fj_out)):
    for k in range(NFCHUNK):
      chunks.append((idx, k * FCHUNK, out))

  n = len(chunks)
  stores = [None] * n
  for k, (idx, off, out) in enumerate(chunks):
    b = k % 2
    if k >= 2:
      stores[k - 2].wait()
    g = pltpu.async_copy(feat_tab.at[idx.at[pl.ds(off, FCHUNK)]],
                         bufs[b], gsems[b])
    g.wait()
    stores[k] = pltpu.async_copy(bufs[b], out.at[pl.ds(base + off, FCHUNK)],
                                 ssems[b])
  for k in range(max(0, n - 2), n):
    stores[k].wait()


def _sc_narrow_body(idx_hbm, tab, out,
                    idx_v, eb0, sem):
  wid = lax.axis_index("s") * NC + lax.axis_index("c")
  base = wid * B_PER_W

  pltpu.sync_copy(idx_hbm.at[pl.ds(base, B_PER_W)], idx_v)

  def body(g, carry):
    v = idx_v[pl.ds(g * 16, 16)]
    for l in range(16):
      pltpu.async_copy(tab.at[pl.ds(v[l], 1)],
                       eb0.at[pl.ds(g * 16 + l, 1)], sem)
    return carry

  lax.fori_loop(0, B_PER_W // 16, body, 0)
  pltpu.make_async_copy(tab.at[pl.ds(0, B_PER_W)], eb0, sem).wait()
  pltpu.sync_copy(eb0, out.at[pl.ds(base, B_PER_W)])


_MESH = plsc.VectorSubcoreMesh(core_axis_name="c", subcore_axis_name="s")


@jax.jit
def _sc_gather(user, item_i, item_j, user_tab, item_tab, feat_tab):
  f32 = jnp.float32
  fi, fj = pl.kernel(
      _sc_feat_body,
      out_type=(
          jax.ShapeDtypeStruct((B, DIM_FEAT), f32),
          jax.ShapeDtypeStruct((B, DIM_FEAT), f32),
      ),
      mesh=_MESH,
      scratch_types=(
          pltpu.VMEM((B_PER_W,), jnp.int32),
          pltpu.VMEM((B_PER_W,), jnp.int32),
          pltpu.VMEM((FCHUNK, DIM_FEAT), f32),
          pltpu.VMEM((FCHUNK, DIM_FEAT), f32),
          pltpu.SemaphoreType.DMA,
          pltpu.SemaphoreType.DMA,
          pltpu.SemaphoreType.DMA,
          pltpu.SemaphoreType.DMA,
      ),
      name="vbpr_sc_feat",
  )(item_i, item_j, feat_tab)

  def narrow(idx, tab, tag):
    return pl.kernel(
        _sc_narrow_body,
        out_type=jax.ShapeDtypeStruct((B, FACTORS), f32),
        mesh=_MESH,
        scratch_types=(
            pltpu.VMEM((B_PER_W,), jnp.int32),
            pltpu.VMEM((B_PER_W, FACTORS), f32),
            pltpu.SemaphoreType.DMA,
        ),
        name="vbpr_sc_narrow_" + tag,
    )(idx, tab)

  user, item_i, item_j = lax.optimization_barrier((user, item_i, item_j, fi))[:3]

  u_emb = narrow(user, user_tab, "u")
  ii_emb = narrow(item_i, item_tab, "i")
  ij_emb = narrow(item_j, item_tab, "j")

  return u_emb, ii_emb, ij_emb, fi, fj


BM = 2048


def _tc_dense_body(u_ref, ii_ref, ij_ref, fi_ref, fj_ref,
                   wv_ref, watt_ref, bvis_ref, wvb_ref, scal_ref, out_ref):
  u = u_ref[...]
  wv = wv_ref[...]
  watt = watt_ref[...]
  bvis = bvis_ref[...]
  wvb = wvb_ref[...]
  b_vbias = scal_ref[0, 0]
  b_att = scal_ref[0, 1]

  def score(item_emb, feat):
    vis = lax.dot_general(feat, wv, (((1,), (1,)), ((), ())),
                          preferred_element_type=jnp.float32) + bvis
    a_item = jnp.tanh(jnp.sum(item_emb * watt, axis=1) + b_att)
    a_vis = jnp.tanh(jnp.sum(vis * watt, axis=1) + b_att)
    e_item = jnp.exp(a_item)
    e_vis = jnp.exp(a_vis)
    denom = e_item + e_vis
    d_item = jnp.sum(u * item_emb, axis=1)
    d_vis = jnp.sum(u * vis, axis=1)
    featb = jnp.sum(feat * wvb, axis=1)
    return (e_item * d_item + e_vis * d_vis) / denom + featb + b_vbias

  out_ref[...] = (score(ii_ref[...], fi_ref[...])
                  - score(ij_ref[...], fj_ref[...]))[:, None]


@jax.jit
def _tc_dense(u_emb, ii_emb, ij_emb, fi, fj, W_vis, w_att, b_vis, w_vbias,
              scalars):
  grid = (B // BM,)
  row = lambda i: (i, 0)
  fixed = lambda i: (0, 0)
  out = pl.pallas_call(
      _tc_dense_body,
      grid=grid,
      in_specs=[
          pl.BlockSpec((BM, FACTORS), row),
          pl.BlockSpec((BM, FACTORS), row),
          pl.BlockSpec((BM, FACTORS), row),
          pl.BlockSpec((BM, DIM_FEAT), row),
          pl.BlockSpec((BM, DIM_FEAT), row),
          pl.BlockSpec((FACTORS, DIM_FEAT), fixed),
          pl.BlockSpec((1, FACTORS), fixed),
          pl.BlockSpec((1, FACTORS), fixed),
          pl.BlockSpec((1, DIM_FEAT), fixed),
          pl.BlockSpec((1, 2), fixed),
      ],
      out_specs=pl.BlockSpec((BM, 1), row),
      out_shape=jax.ShapeDtypeStruct((B, 1), jnp.float32),
      name="vbpr_tc_dense",
  )(u_emb, ii_emb, ij_emb, fi, fj, W_vis, w_att, b_vis, w_vbias, scalars)
  return out.reshape(B)


def kernel(user, item_i, item_j, user_table, item_table, item_features,
           W_vis, b_vis, w_vbias, b_vbias, w_att, b_att):
  user = user.astype(jnp.int32)
  item_i = item_i.astype(jnp.int32)
  item_j = item_j.astype(jnp.int32)
  u_emb, ii_emb, ij_emb, fi, fj = _sc_gather(
      user, item_i, item_j, user_table, item_table, item_features)
  scalars = jnp.stack([b_vbias, b_att]).reshape(1, 2).astype(jnp.float32)
  return _tc_dense(u_emb, ii_emb, ij_emb, fi, fj,
                   W_vis, w_att.reshape(1, FACTORS),
                   b_vis.reshape(1, FACTORS), w_vbias.reshape(1, DIM_FEAT),
                   scalars)
